# Initial kernel scaffold; baseline (speedup 1.0000x reference)
import jax
import jax.numpy as jnp
from jax.experimental import pallas as pl


def kernel(node_feats, edge_feats, edge_index, W_src_gate, b_src_gate, W_dst_gate, b_dst_gate, W_edge_gate, b_edge_gate, W_src_update, b_src_update, W_dst_update, b_dst_update, gamma_nodes, beta_nodes, gamma_edges, beta_edges):
    raise NotImplementedError("write your pallas kernel here")


# speedup vs baseline: 1.4075x; 1.4075x over previous
import functools

import jax
import jax.numpy as jnp
from jax import lax
from jax.experimental import pallas as pl
from jax.experimental.pallas import tpu as pltpu
from jax.experimental.pallas import tpu_sc as plsc

N = 10000
E = 320000
D = 128
H = 64
NSUB = 16
NCORE = 2
EPW = E // NSUB
C = 80
NCH = EPW // C

BE = 2000


def _node_proj_body(x_ref, wsg, bsg, wdg, bdg, wdu, bdu, wsu, bsu,
                    tsg, tdgdu0, tdgdu1, psu):
    x = x_ref[...]
    dn = (((1,), (1,)), ((), ()))
    sg = lax.dot_general(x, wsg[...], dn, preferred_element_type=jnp.float32) + bsg[...]
    dg = lax.dot_general(x, wdg[...], dn, preferred_element_type=jnp.float32) + bdg[...]
    du = lax.dot_general(x, wdu[...], dn, preferred_element_type=jnp.float32) + bdu[...]
    su = lax.dot_general(x, wsu[...], dn, preferred_element_type=jnp.float32) + bsu[...]
    tsg[...] = sg
    tdgdu0[...] = jnp.concatenate([dg[:, :H], du[:, :H]], axis=-1)
    tdgdu1[...] = jnp.concatenate([dg[:, H:], du[:, H:]], axis=-1)
    psu[...] = su


def _node_proj(x, wsg, bsg, wdg, bdg, wdu, bdu, wsu, bsu):
    full = jax.ShapeDtypeStruct((N, D), jnp.float32)
    return pl.pallas_call(
        _node_proj_body,
        out_shape=(full, full, full, full),
    )(x, wsg, bsg, wdg, bdg, wdu, bdu, wsu, bsu)


def _edge_proj_body(ef_ref, w, b, ep0, ep1):
    t = lax.dot_general(ef_ref[...], w[...], (((1,), (1,)), ((), ())),
                        preferred_element_type=jnp.float32) + b[...]
    ep0[...] = t[:, :H]
    ep1[...] = t[:, H:]


def _edge_proj(ef, w, b):
    nblk = E // BE
    half = jax.ShapeDtypeStruct((E, H), jnp.float32)
    return pl.pallas_call(
        _edge_proj_body,
        grid=(nblk,),
        in_specs=[
            pl.BlockSpec((BE, D), lambda i: (i, 0)),
            pl.BlockSpec((D, D), lambda i: (0, 0)),
            pl.BlockSpec((1, D), lambda i: (0, 0)),
        ],
        out_specs=(
            pl.BlockSpec((BE, H), lambda i: (i, 0)),
            pl.BlockSpec((BE, H), lambda i: (i, 0)),
        ),
        out_shape=(half, half),
    )(ef, w, b)


def _sc_body(ep0, ep1, tsg, tdgdu0, tdgdu1, idx_i, idx_j, zeros_hbm,
             y0, y1, sc_comb, stats,
             ii, ij, sg_b, dgdu_b, y_b, msig_b, stat_b,
             s_acc, sem):
    c = lax.axis_index("c")
    s = lax.axis_index("s")

    @pl.when(s == 0)
    def _():
        pltpu.sync_copy(zeros_hbm, s_acc)

    plsc.subcore_barrier()

    base0 = s * EPW

    def chunk_body(k, carry):
        base = base0 + k * C
        d_ii = pltpu.async_copy(idx_i.at[pl.ds(base, C)], ii, sem)
        d_ij = pltpu.async_copy(idx_j.at[pl.ds(base, C)], ij, sem)
        d_ii.wait()
        d_ij.wait()

        @pl.when(c == 0)
        def _():
            d_ep = pltpu.async_copy(ep0.at[pl.ds(base, C)], y_b, sem)
            d_sg = pltpu.async_copy(tsg.at[ii], sg_b, sem)
            d_dd = pltpu.async_copy(tdgdu0.at[ij], dgdu_b, sem)
            d_ep.wait()
            d_sg.wait()
            d_dd.wait()

        @pl.when(c == 1)
        def _():
            d_ep = pltpu.async_copy(ep1.at[pl.ds(base, C)], y_b, sem)
            d_sg = pltpu.async_copy(tsg.at[ii], sg_b, sem)
            d_dd = pltpu.async_copy(tdgdu1.at[ij], dgdu_b, sem)
            d_ep.wait()
            d_sg.wait()
            d_dd.wait()

        coff = c * H

        def row(i, cr):
            out = list(cr)
            for j in range(H // 16):
                sl = pl.ds(j * 16, 16)
                yv = (y_b[i, sl] + sg_b[i, pl.ds(coff + j * 16, 16)]
                      + dgdu_b[i, sl])
                y_b[i, sl] = yv
                sgm = 1.0 / (1.0 + jnp.exp(-yv))
                msig_b[i, pl.ds(H + j * 16, 16)] = sgm
                msig_b[i, sl] = sgm * dgdu_b[i, pl.ds(H + j * 16, 16)]
                out[j] = cr[j] + yv
                out[4 + j] = cr[4 + j] + yv * yv
            return tuple(out)

        carry = lax.fori_loop(0, C, row, carry)

        @pl.when(c == 0)
        def _():
            pltpu.sync_copy(y_b, y0.at[pl.ds(base, C)])

        @pl.when(c == 1)
        def _():
            pltpu.sync_copy(y_b, y1.at[pl.ds(base, C)])

        pltpu.sync_copy(msig_b, s_acc.at[ii], add=True)
        return carry

    zero = jnp.zeros((16,), jnp.float32)
    carry = lax.fori_loop(0, NCH, chunk_body, (zero,) * 8)

    for j in range(H // 16):
        stat_b[0, pl.ds(j * 16, 16)] = carry[j]
        stat_b[1, pl.ds(j * 16, 16)] = carry[4 + j]
    pltpu.sync_copy(stat_b, stats.at[c, s])

    plsc.subcore_barrier()

    @pl.when(s == 0)
    def _():
        pltpu.sync_copy(s_acc, sc_comb.at[c])


def _sc_gather_scatter(ep0, ep1, tsg, tdgdu0, tdgdu1, idx_i, idx_j, zeros_hbm):
    mesh = plsc.VectorSubcoreMesh(core_axis_name="c", subcore_axis_name="s",
                                  num_cores=NCORE, num_subcores=NSUB)
    f32 = jnp.float32
    fn = pl.kernel(
        _sc_body,
        out_type=[
            jax.ShapeDtypeStruct((E, H), f32),
            jax.ShapeDtypeStruct((E, H), f32),
            jax.ShapeDtypeStruct((NCORE, N, D), f32),
            jax.ShapeDtypeStruct((NCORE, NSUB, 2, H), f32),
        ],
        mesh=mesh,
        scratch_types=[
            pltpu.VMEM((C,), jnp.int32),
            pltpu.VMEM((C,), jnp.int32),
            pltpu.VMEM((C, D), f32),
            pltpu.VMEM((C, D), f32),
            pltpu.VMEM((C, H), f32),
            pltpu.VMEM((C, D), f32),
            pltpu.VMEM((2, H), f32),
            pltpu.VMEM_SHARED((N, D), f32),
            pltpu.SemaphoreType.DMA,
        ],
    )
    return fn(ep0, ep1, tsg, tdgdu0, tdgdu1, idx_i, idx_j, zeros_hbm)


def _node_out_body(sc_comb, psu, nf, stats, gn, bn, ge2, be2, x_out, bnp):
    s1 = jnp.concatenate([sc_comb[0, :, :H], sc_comb[1, :, :H]], axis=-1)
    s2 = jnp.concatenate([sc_comb[0, :, H:], sc_comb[1, :, H:]], axis=-1)
    h = s1 / (s2 + 1e-6)
    xlin = psu[...] + h
    mu = jnp.mean(xlin, axis=0, keepdims=True)
    var = jnp.mean((xlin - mu) ** 2, axis=0, keepdims=True)
    xn = gn[...] * (xlin - mu) / jnp.sqrt(var + 1e-5) + bn[...]
    x = xn * jax.nn.sigmoid(xn)
    x_out[...] = nf[...] + x

    st = stats[...]
    sums = jnp.sum(st[:, :, 0, :], axis=1)
    sqs = jnp.sum(st[:, :, 1, :], axis=1)
    mu_e = sums / float(E)
    var_e = sqs / float(E) - mu_e * mu_e
    rstd = 1.0 / jnp.sqrt(var_e + 1e-5)
    a = ge2[...] * rstd
    b = be2[...] - ge2[...] * mu_e * rstd
    bnp[0] = a
    bnp[1] = b


def _node_out(sc_comb, psu, nf, stats, gn, bn, ge2, be2):
    return pl.pallas_call(
        _node_out_body,
        out_shape=(
            jax.ShapeDtypeStruct((N, D), jnp.float32),
            jax.ShapeDtypeStruct((2, NCORE, H), jnp.float32),
        ),
    )(sc_comb, psu, nf, stats, gn, bn, ge2, be2)


def _edge_out_body(y0, y1, ef, bnp, yout):
    a = bnp[...]
    t0 = y0[...] * a[0, 0][None, :] + a[1, 0][None, :]
    t1 = y1[...] * a[0, 1][None, :] + a[1, 1][None, :]
    t = jnp.concatenate([t0, t1], axis=-1)
    yout[...] = ef[...] + t * jax.nn.sigmoid(t)


def _edge_out(y0, y1, ef, bnp):
    nblk = E // BE
    return pl.pallas_call(
        _edge_out_body,
        grid=(nblk,),
        in_specs=[
            pl.BlockSpec((BE, H), lambda i: (i, 0)),
            pl.BlockSpec((BE, H), lambda i: (i, 0)),
            pl.BlockSpec((BE, D), lambda i: (i, 0)),
            pl.BlockSpec((2, NCORE, H), lambda i: (0, 0, 0)),
        ],
        out_specs=pl.BlockSpec((BE, D), lambda i: (i, 0)),
        out_shape=jax.ShapeDtypeStruct((E, D), jnp.float32),
    )(y0, y1, ef, bnp)


def kernel(node_feats, edge_feats, edge_index, W_src_gate, b_src_gate, W_dst_gate,
           b_dst_gate, W_edge_gate, b_edge_gate, W_src_update, b_src_update,
           W_dst_update, b_dst_update, gamma_nodes, beta_nodes, gamma_edges,
           beta_edges):
    f32 = jnp.float32
    idx_i = edge_index[0]
    idx_j = edge_index[1]

    bsg = b_src_gate.reshape(1, D)
    bdg = b_dst_gate.reshape(1, D)
    bdu = b_dst_update.reshape(1, D)
    bsu = b_src_update.reshape(1, D)
    beg = b_edge_gate.reshape(1, D)
    gn = gamma_nodes.reshape(1, D)
    bn = beta_nodes.reshape(1, D)
    ge2 = gamma_edges.reshape(NCORE, H)
    be2 = beta_edges.reshape(NCORE, H)

    tsg, tdgdu0, tdgdu1, psu = _node_proj(
        node_feats, W_src_gate, bsg, W_dst_gate, bdg, W_dst_update, bdu,
        W_src_update, bsu)
    ep0, ep1 = _edge_proj(edge_feats, W_edge_gate, beg)

    zeros_hbm = jnp.zeros((N, D), f32)
    y0, y1, sc_comb, stats = _sc_gather_scatter(
        ep0, ep1, tsg, tdgdu0, tdgdu1, idx_i, idx_j, zeros_hbm)

    x_out, bnp = _node_out(sc_comb, psu, node_feats, stats, gn, bn, ge2, be2)
    y_out = _edge_out(y0, y1, edge_feats, bnp)
    return (x_out, y_out)
